# baseline (speedup 1.0000x reference)
import functools
import math

import jax
import jax.numpy as jnp
from jax.experimental import pallas as pl
from jax.experimental.pallas import tpu as pltpu

T, D, E = 2048, 2048, 16
C = int(2 * math.ceil(T / (E // 4)))
TB = 128


def _route_body(x_ref, wg_ref, flat1_ref, flat2_ref, g1_ref, g2_ref, laux_ref):
    x = x_ref[...]
    wg = wg_ref[...]
    logits = jax.lax.dot_general(
        x, wg, (((1,), (1,)), ((), ())), preferred_element_type=jnp.float32
    )
    m = jnp.max(logits, axis=1, keepdims=True)
    ex = jnp.exp(logits - m)
    gates = ex / jnp.sum(ex, axis=1, keepdims=True)

    e_iota = jax.lax.broadcasted_iota(jnp.int32, (T, E), 1)
    e1 = jnp.argmax(gates, axis=1, keepdims=True).astype(jnp.int32)
    mask1 = e_iota == e1
    gates_m1 = jnp.where(mask1, -jnp.inf, gates)
    e2 = jnp.argmax(gates_m1, axis=1, keepdims=True).astype(jnp.int32)
    mask2 = e_iota == e2

    mask1i = mask1.astype(jnp.int32)
    mask2i = mask2.astype(jnp.int32)

    cum1 = mask1i
    cum2 = mask2i
    k = 1
    while k < T:
        z = jnp.zeros((k, E), jnp.int32)
        cum1 = cum1 + jnp.concatenate([z, cum1[: T - k]], axis=0)
        cum2 = cum2 + jnp.concatenate([z, cum2[: T - k]], axis=0)
        k *= 2
    cnt1 = cum1[T - 1 :, :]
    loc1 = cum1 - 1
    loc2 = cum2 - 1 + cnt1

    mask1f = mask1i.astype(jnp.float32)
    mask2f = mask2i.astype(jnp.float32)
    loc1_s = jnp.sum(loc1 * mask1i, axis=1, keepdims=True)
    loc2_s = jnp.sum(loc2 * mask2i, axis=1, keepdims=True)
    g1 = jnp.sum(gates * mask1f, axis=1, keepdims=True)
    g2 = jnp.sum(gates * mask2f, axis=1, keepdims=True)
    kept1 = loc1_s < C
    kept2 = loc2_s < C
    g1 = g1 * kept1.astype(jnp.float32)
    g2 = g2 * kept2.astype(jnp.float32)
    denom = jnp.maximum(g1 + g2, jnp.finfo(jnp.float32).eps)
    g1_ref[...] = g1 / denom
    g2_ref[...] = g2 / denom
    flat1_ref[...] = jnp.where(kept1, e1 * C + loc1_s, -1)
    flat2_ref[...] = jnp.where(kept2, e2 * C + loc2_s, -1)

    me = jnp.mean(gates, axis=0, keepdims=True)
    ce = cnt1.astype(jnp.float32) / T
    laux_ref[...] = jnp.sum(me * ce, keepdims=True) / (E // 4) * E * E


def _route(x, wg, *, interpret=False):
    return pl.pallas_call(
        _route_body,
        out_shape=[
            jax.ShapeDtypeStruct((T, 1), jnp.int32),
            jax.ShapeDtypeStruct((T, 1), jnp.int32),
            jax.ShapeDtypeStruct((T, 1), jnp.float32),
            jax.ShapeDtypeStruct((T, 1), jnp.float32),
            jax.ShapeDtypeStruct((1, 1), jnp.float32),
        ],
        interpret=interpret,
    )(x, wg)


def _write_body(flat1_ref, flat2_ref, g1_ref, g2_ref, cw_ref, mask_ref):
    f1 = flat1_ref[...]
    f2 = flat2_ref[...]
    g1 = g1_ref[...]
    g2 = g2_ref[...]
    k = jax.lax.broadcasted_iota(jnp.int32, (TB, E * C), 1)
    cw = jnp.where(k == f1, g1, 0.0) + jnp.where(k == f2, g2, 0.0)
    cw_ref[...] = cw
    mask_ref[...] = cw != 0.0


def _write(flat1, flat2, g1, g2, *, interpret=False):
    grid = (T // TB,)
    tok = pl.BlockSpec((TB, 1), lambda i: (i, 0))
    big = pl.BlockSpec((TB, E * C), lambda i: (i, 0))
    return pl.pallas_call(
        _write_body,
        grid=grid,
        in_specs=[tok, tok, tok, tok],
        out_specs=[big, big],
        out_shape=[
            jax.ShapeDtypeStruct((T, E * C), jnp.float32),
            jax.ShapeDtypeStruct((T, E * C), jnp.bool_),
        ],
        interpret=interpret,
    )(flat1, flat2, g1, g2)


def kernel(input, wg):
    flat1, flat2, g1, g2, laux = _route(input, wg)
    cw, mask = _write(flat1, flat2, g1, g2)
    return (
        laux.reshape(()),
        cw.reshape(T, E, C),
        mask.reshape(T, E, C),
    )


# speedup vs baseline: 2.0036x; 2.0036x over previous
import functools
import math

import jax
import jax.numpy as jnp
from jax.experimental import pallas as pl
from jax.experimental.pallas import tpu as pltpu

T, D, E = 2048, 2048, 16
C = int(2 * math.ceil(T / (E // 4)))
TB = 128


def _route_body(x_ref, wg_ref, flat1_ref, flat2_ref, g1_ref, g2_ref, laux_ref):
    x = x_ref[...]
    wg = wg_ref[...]
    logits = jax.lax.dot_general(
        x, wg, (((1,), (1,)), ((), ())), preferred_element_type=jnp.float32
    )
    m = jnp.max(logits, axis=1, keepdims=True)
    ex = jnp.exp(logits - m)
    gates = ex / jnp.sum(ex, axis=1, keepdims=True)

    e_iota = jax.lax.broadcasted_iota(jnp.int32, (T, E), 1)
    e1 = jnp.argmax(gates, axis=1, keepdims=True).astype(jnp.int32)
    mask1 = e_iota == e1
    gates_m1 = jnp.where(mask1, -jnp.inf, gates)
    e2 = jnp.argmax(gates_m1, axis=1, keepdims=True).astype(jnp.int32)
    mask2 = e_iota == e2

    mask1i = mask1.astype(jnp.int32)
    mask2i = mask2.astype(jnp.int32)

    cum1 = mask1i
    cum2 = mask2i
    k = 1
    while k < T:
        z = jnp.zeros((k, E), jnp.int32)
        cum1 = cum1 + jnp.concatenate([z, cum1[: T - k]], axis=0)
        cum2 = cum2 + jnp.concatenate([z, cum2[: T - k]], axis=0)
        k *= 2
    cnt1 = cum1[T - 1 :, :]
    loc1 = cum1 - 1
    loc2 = cum2 - 1 + cnt1

    mask1f = mask1i.astype(jnp.float32)
    mask2f = mask2i.astype(jnp.float32)
    loc1_s = jnp.sum(loc1 * mask1i, axis=1, keepdims=True)
    loc2_s = jnp.sum(loc2 * mask2i, axis=1, keepdims=True)
    g1 = jnp.sum(gates * mask1f, axis=1, keepdims=True)
    g2 = jnp.sum(gates * mask2f, axis=1, keepdims=True)
    kept1 = loc1_s < C
    kept2 = loc2_s < C
    g1 = g1 * kept1.astype(jnp.float32)
    g2 = g2 * kept2.astype(jnp.float32)
    denom = jnp.maximum(g1 + g2, jnp.finfo(jnp.float32).eps)
    g1_ref[...] = g1 / denom
    g2_ref[...] = g2 / denom
    flat1_ref[...] = jnp.where(kept1, e1 * C + loc1_s, -1)
    flat2_ref[...] = jnp.where(kept2, e2 * C + loc2_s, -1)

    me = jnp.mean(gates, axis=0, keepdims=True)
    ce = cnt1.astype(jnp.float32) / T
    laux_ref[...] = jnp.sum(me * ce, keepdims=True) / (E // 4) * E * E


def _route(x, wg, *, interpret=False):
    return pl.pallas_call(
        _route_body,
        out_shape=[
            jax.ShapeDtypeStruct((T, 1), jnp.int32),
            jax.ShapeDtypeStruct((T, 1), jnp.int32),
            jax.ShapeDtypeStruct((T, 1), jnp.float32),
            jax.ShapeDtypeStruct((T, 1), jnp.float32),
            jax.ShapeDtypeStruct((1, 1), jnp.float32),
        ],
        interpret=interpret,
    )(x, wg)


def _write_body(flat1_ref, flat2_ref, g1_ref, g2_ref, cw_ref, mask_ref):
    f1 = flat1_ref[...]
    f2 = flat2_ref[...]
    g1 = g1_ref[...]
    g2 = g2_ref[...]
    k = jax.lax.broadcasted_iota(jnp.int32, (TB, E, C), 1) * C + (
        jax.lax.broadcasted_iota(jnp.int32, (TB, E, C), 2)
    )
    cw = jnp.where(k == f1, g1, 0.0) + jnp.where(k == f2, g2, 0.0)
    cw_ref[...] = cw
    mask_ref[...] = cw != 0.0


def _write(flat1, flat2, g1, g2, *, interpret=False):
    grid = (T // TB,)
    tok = pl.BlockSpec((TB, 1, 1), lambda i: (i, 0, 0))
    big = pl.BlockSpec((TB, E, C), lambda i: (i, 0, 0))
    return pl.pallas_call(
        _write_body,
        grid=grid,
        in_specs=[tok, tok, tok, tok],
        out_specs=[big, big],
        out_shape=[
            jax.ShapeDtypeStruct((T, E, C), jnp.float32),
            jax.ShapeDtypeStruct((T, E, C), jnp.bool_),
        ],
        interpret=interpret,
    )(flat1, flat2, g1, g2)


def kernel(input, wg):
    flat1, flat2, g1, g2, laux = _route(input, wg)
    cw, mask = _write(
        flat1.reshape(T, 1, 1),
        flat2.reshape(T, 1, 1),
        g1.reshape(T, 1, 1),
        g2.reshape(T, 1, 1),
    )
    return (laux.reshape(()), cw, mask)


# speedup vs baseline: 2.9698x; 1.4822x over previous
import math

import jax
import jax.numpy as jnp
from jax.experimental import pallas as pl
from jax.experimental.pallas import tpu as pltpu

T, D, E = 2048, 2048, 16
C = int(2 * math.ceil(T / (E // 4)))
TB = 256


def _route_body(x_ref, wg_ref, flat1_ref, flat2_ref, g1_ref, g2_ref, laux_ref):
    x = x_ref[...]
    wg = wg_ref[...]
    logits = jax.lax.dot_general(
        x, wg, (((1,), (1,)), ((), ())), preferred_element_type=jnp.float32
    )
    m = jnp.max(logits, axis=1, keepdims=True)
    ex = jnp.exp(logits - m)
    gates = ex / jnp.sum(ex, axis=1, keepdims=True)

    e_iota = jax.lax.broadcasted_iota(jnp.int32, (T, E), 1)
    e1 = jnp.argmax(gates, axis=1, keepdims=True).astype(jnp.int32)
    mask1 = e_iota == e1
    gates_m1 = jnp.where(mask1, -jnp.inf, gates)
    e2 = jnp.argmax(gates_m1, axis=1, keepdims=True).astype(jnp.int32)
    mask2 = e_iota == e2

    mask1i = mask1.astype(jnp.int32)
    mask2i = mask2.astype(jnp.int32)

    cum1 = mask1i
    cum2 = mask2i
    k = 1
    while k < T:
        z = jnp.zeros((k, E), jnp.int32)
        cum1 = cum1 + jnp.concatenate([z, cum1[: T - k]], axis=0)
        cum2 = cum2 + jnp.concatenate([z, cum2[: T - k]], axis=0)
        k *= 2
    cnt1 = cum1[T - 1 :, :]
    loc1 = cum1 - 1
    loc2 = cum2 - 1 + cnt1

    mask1f = mask1i.astype(jnp.float32)
    mask2f = mask2i.astype(jnp.float32)
    loc1_s = jnp.sum(loc1 * mask1i, axis=1, keepdims=True)
    loc2_s = jnp.sum(loc2 * mask2i, axis=1, keepdims=True)
    g1 = jnp.sum(gates * mask1f, axis=1, keepdims=True)
    g2 = jnp.sum(gates * mask2f, axis=1, keepdims=True)
    kept1 = loc1_s < C
    kept2 = loc2_s < C
    g1 = g1 * kept1.astype(jnp.float32)
    g2 = g2 * kept2.astype(jnp.float32)
    denom = jnp.maximum(g1 + g2, jnp.finfo(jnp.float32).eps)
    g1_ref[...] = g1 / denom
    g2_ref[...] = g2 / denom
    flat1_ref[...] = jnp.where(kept1, e1 * C + loc1_s, -1)
    flat2_ref[...] = jnp.where(kept2, e2 * C + loc2_s, -1)

    me = jnp.mean(gates, axis=0, keepdims=True)
    ce = cnt1.astype(jnp.float32) / T
    laux_ref[...] = jnp.sum(me * ce, keepdims=True) / (E // 4) * E * E


def _route(x, wg, *, interpret=False):
    return pl.pallas_call(
        _route_body,
        out_shape=[
            jax.ShapeDtypeStruct((T, 1), jnp.int32),
            jax.ShapeDtypeStruct((T, 1), jnp.int32),
            jax.ShapeDtypeStruct((T, 1), jnp.float32),
            jax.ShapeDtypeStruct((T, 1), jnp.float32),
            jax.ShapeDtypeStruct((1, 1), jnp.float32),
        ],
        interpret=interpret,
    )(x, wg)


def _write_body(flat1_ref, flat2_ref, g1_ref, g2_ref, cw_ref, mask_ref):
    f1 = flat1_ref[...]
    f2 = flat2_ref[...]
    g1 = g1_ref[...]
    g2 = g2_ref[...]
    k = jax.lax.broadcasted_iota(jnp.int32, (TB, E, C), 1) * C + (
        jax.lax.broadcasted_iota(jnp.int32, (TB, E, C), 2)
    )
    cw = jnp.where(k == f1, g1, 0.0) + jnp.where(k == f2, g2, 0.0)
    cw_ref[...] = cw
    mask_ref[...] = (cw != 0.0).astype(jnp.int8)


def _write(flat1, flat2, g1, g2, *, interpret=False):
    grid = (T // TB,)
    tok = pl.BlockSpec((TB, 1, 1), lambda i: (i, 0, 0))
    big = pl.BlockSpec((TB, E, C), lambda i: (i, 0, 0))
    return pl.pallas_call(
        _write_body,
        grid=grid,
        in_specs=[tok, tok, tok, tok],
        out_specs=[big, big],
        out_shape=[
            jax.ShapeDtypeStruct((T, E, C), jnp.float32),
            jax.ShapeDtypeStruct((T, E, C), jnp.int8),
        ],
        interpret=interpret,
    )(flat1, flat2, g1, g2)


def kernel(input, wg):
    flat1, flat2, g1, g2, laux = _route(input, wg)
    cw, mask8 = _write(
        flat1.reshape(T, 1, 1),
        flat2.reshape(T, 1, 1),
        g1.reshape(T, 1, 1),
        g2.reshape(T, 1, 1),
    )
    return (laux.reshape(()), cw, mask8.view(jnp.bool_))
